# Initial kernel scaffold; baseline (speedup 1.0000x reference)
import jax
import jax.numpy as jnp
from jax.experimental import pallas as pl


def kernel(x, edge_index, batch, classes_emb, neigh_emb, Ws, bs, gammas, betas, Wout):
    raise NotImplementedError("write your pallas kernel here")


# speedup vs baseline: 11.9277x; 11.9277x over previous
import functools

import jax
import jax.numpy as jnp
from jax import lax
from jax.experimental import pallas as pl
from jax.experimental.pallas import tpu as pltpu
from jax.experimental.pallas import tpu_sc as plsc

N = 10000
E = 320000
H = 128
IN_DIM = 32
NB = 100
LAYERS = 4

NC = 2
NS = 16
NW = NC * NS

K = 128
CW = -(-E // (K * NW))
C_TOT = CW * NW
E_PAD = C_TOT * K
N_ACC = 10240
RPT = N_ACC // NS

_mesh = plsc.VectorSubcoreMesh(
    core_axis_name="c", subcore_axis_name="s", num_cores=NC, num_subcores=NS
)


def _zero_buf(buf, rows, width):
    def zrow(i, carry):
        for j in range(width // 16):
            buf[i, pl.ds(j * 16, 16)] = jnp.zeros((16,), jnp.float32)
        return carry
    lax.fori_loop(0, rows, zrow, 0)


def _fill_ones(buf, rows, width):
    def frow(i, carry):
        for j in range(width // 16):
            buf[i, pl.ds(j * 16, 16)] = jnp.ones((16,), jnp.float32)
        return carry
    lax.fori_loop(0, rows, frow, 0)


@functools.partial(
    pl.kernel,
    out_type=jax.ShapeDtypeStruct((NC, N_ACC, 16), jnp.float32),
    mesh=_mesh,
    scratch_types=[
        pltpu.VMEM((K,), jnp.int32),
        pltpu.VMEM((K, 16), jnp.float32),
        pltpu.VMEM_SHARED((N_ACC, 16), jnp.float32),
    ],
)
def _deg_kernel(dsts_hbm, out_hbm, dst_v, ones_v, acc):
    c = lax.axis_index("c")
    s = lax.axis_index("s")
    _zero_buf(ones_v, K, 16)
    r0 = s * RPT
    for w in range(RPT // K):
        pltpu.sync_copy(ones_v, acc.at[pl.ds(r0 + w * K, K)])
    _fill_ones(ones_v, K, 16)
    plsc.subcore_barrier()

    base = (c * NS + s) * CW

    def body(t, carry):
        pltpu.sync_copy(dsts_hbm.at[base + t], dst_v)
        pltpu.sync_copy(ones_v, acc.at[dst_v], add=True)
        return carry

    lax.fori_loop(0, CW, body, 0)
    plsc.subcore_barrier()
    for w in range(RPT // K):
        sl = pl.ds(r0 + w * K, K)
        pltpu.sync_copy(acc.at[sl], out_hbm.at[c].at[sl])


@functools.partial(
    pl.kernel,
    out_type=jax.ShapeDtypeStruct((NC, N_ACC, H), jnp.float32),
    mesh=_mesh,
    scratch_types=[
        pltpu.VMEM((K,), jnp.int32),
        pltpu.VMEM((K,), jnp.int32),
        pltpu.VMEM((K, H), jnp.float32),
        pltpu.VMEM_SHARED((N_ACC, H), jnp.float32),
        pltpu.SemaphoreType.DMA,
    ],
)
def _agg_kernel(hw_hbm, srcs_hbm, dsts_hbm, out_hbm, src_v, dst_v, rows_v, acc, sem):
    c = lax.axis_index("c")
    s = lax.axis_index("s")
    _zero_buf(rows_v, K, H)
    r0 = s * RPT
    for w in range(RPT // K):
        pltpu.sync_copy(rows_v, acc.at[pl.ds(r0 + w * K, K)])
    plsc.subcore_barrier()

    base = (c * NS + s) * CW

    def body(t, carry):
        cid = base + t
        pltpu.sync_copy(srcs_hbm.at[cid], src_v)
        pltpu.sync_copy(dsts_hbm.at[cid], dst_v)
        pltpu.async_copy(hw_hbm.at[src_v], rows_v, sem).wait()
        pltpu.sync_copy(rows_v, acc.at[dst_v], add=True)
        return carry

    lax.fori_loop(0, CW, body, 0)
    plsc.subcore_barrier()
    for w in range(RPT // K):
        sl = pl.ds(r0 + w * K, K)
        pltpu.sync_copy(acc.at[sl], out_hbm.at[c].at[sl])


R = 1000
GRID = N // R


def _pre_body(x0_ref, x1_ref, degp_ref, ce_ref, ne_ref, w0_ref,
              h_ref, dinv_ref, hw_ref):
    x0 = x0_ref[...]
    x1 = x1_ref[...]
    oh0 = (x0 == lax.broadcasted_iota(jnp.int32, (R, IN_DIM), 1)).astype(jnp.float32)
    oh1 = (x1 == lax.broadcasted_iota(jnp.int32, (R, IN_DIM + 1), 1)).astype(jnp.float32)
    h = (jnp.dot(oh0, ce_ref[...], preferred_element_type=jnp.float32)
         + jnp.dot(oh1, ne_ref[...], preferred_element_type=jnp.float32))
    deg = degp_ref[0, :, 0:1] + degp_ref[1, :, 0:1] + 1.0
    dinv = lax.rsqrt(deg)
    h_ref[...] = h
    dinv_ref[...] = dinv
    hw_ref[...] = dinv * jnp.dot(h, w0_ref[...], preferred_element_type=jnp.float32)


_pre_call = pl.pallas_call(
    _pre_body,
    grid=(GRID,),
    in_specs=[
        pl.BlockSpec((R, 1), lambda i: (i, 0)),
        pl.BlockSpec((R, 1), lambda i: (i, 0)),
        pl.BlockSpec((NC, R, 16), lambda i: (0, i, 0)),
        pl.BlockSpec((IN_DIM, H), lambda i: (0, 0)),
        pl.BlockSpec((IN_DIM + 1, H), lambda i: (0, 0)),
        pl.BlockSpec((H, H), lambda i: (0, 0)),
    ],
    out_specs=[
        pl.BlockSpec((R, H), lambda i: (i, 0)),
        pl.BlockSpec((R, 1), lambda i: (i, 0)),
        pl.BlockSpec((R, H), lambda i: (i, 0)),
    ],
    out_shape=[
        jax.ShapeDtypeStruct((N, H), jnp.float32),
        jax.ShapeDtypeStruct((N, 1), jnp.float32),
        jax.ShapeDtypeStruct((N, H), jnp.float32),
    ],
)


def _layer_body(h_ref, hw_ref, p_ref, dinv_ref, b_ref, g_ref, be_ref, wn_ref,
                ho_ref, hwo_ref):
    dinv = dinv_ref[...]
    agg = p_ref[0] + p_ref[1] + hw_ref[...]
    conv = dinv * agg + b_ref[...]
    hn = jnp.maximum(conv, 0.0)
    sres = h_ref[...] + hn
    m = jnp.mean(sres, axis=-1, keepdims=True)
    d = sres - m
    v = jnp.mean(d * d, axis=-1, keepdims=True)
    hnew = d * lax.rsqrt(v + 1e-5) * g_ref[...] + be_ref[...]
    ho_ref[...] = hnew
    hwo_ref[...] = dinv * jnp.dot(hnew, wn_ref[...], preferred_element_type=jnp.float32)


_layer_call = pl.pallas_call(
    _layer_body,
    grid=(GRID,),
    in_specs=[
        pl.BlockSpec((R, H), lambda i: (i, 0)),
        pl.BlockSpec((R, H), lambda i: (i, 0)),
        pl.BlockSpec((NC, R, H), lambda i: (0, i, 0)),
        pl.BlockSpec((R, 1), lambda i: (i, 0)),
        pl.BlockSpec((1, H), lambda i: (0, 0)),
        pl.BlockSpec((1, H), lambda i: (0, 0)),
        pl.BlockSpec((1, H), lambda i: (0, 0)),
        pl.BlockSpec((H, H), lambda i: (0, 0)),
    ],
    out_specs=[
        pl.BlockSpec((R, H), lambda i: (i, 0)),
        pl.BlockSpec((R, H), lambda i: (i, 0)),
    ],
    out_shape=[
        jax.ShapeDtypeStruct((N, H), jnp.float32),
        jax.ShapeDtypeStruct((N, H), jnp.float32),
    ],
)


def _last_body(h_ref, hw_ref, p_ref, dinv_ref, b_ref, g_ref, be_ref, ho_ref):
    dinv = dinv_ref[...]
    agg = p_ref[0] + p_ref[1] + hw_ref[...]
    conv = dinv * agg + b_ref[...]
    hn = jnp.maximum(conv, 0.0)
    sres = h_ref[...] + hn
    m = jnp.mean(sres, axis=-1, keepdims=True)
    d = sres - m
    v = jnp.mean(d * d, axis=-1, keepdims=True)
    ho_ref[...] = d * lax.rsqrt(v + 1e-5) * g_ref[...] + be_ref[...]


_last_call = pl.pallas_call(
    _last_body,
    grid=(GRID,),
    in_specs=[
        pl.BlockSpec((R, H), lambda i: (i, 0)),
        pl.BlockSpec((R, H), lambda i: (i, 0)),
        pl.BlockSpec((NC, R, H), lambda i: (0, i, 0)),
        pl.BlockSpec((R, 1), lambda i: (i, 0)),
        pl.BlockSpec((1, H), lambda i: (0, 0)),
        pl.BlockSpec((1, H), lambda i: (0, 0)),
        pl.BlockSpec((1, H), lambda i: (0, 0)),
    ],
    out_specs=[pl.BlockSpec((R, H), lambda i: (i, 0))],
    out_shape=[jax.ShapeDtypeStruct((N, H), jnp.float32)],
)


def _logits_body(roots_ref, wout_ref, bz_ref, out_ref):
    out_ref[...] = (
        jnp.dot(roots_ref[...], wout_ref[...], preferred_element_type=jnp.float32)
        + bz_ref[...]
    )


_logits_call = pl.pallas_call(
    _logits_body,
    out_shape=jax.ShapeDtypeStruct((NB, IN_DIM), jnp.float32),
)


def kernel(x, edge_index, batch, classes_emb, neigh_emb, Ws, bs, gammas, betas, Wout):
    src = edge_index[0]
    dst = edge_index[1]
    pad = E_PAD - E
    pad_src = jnp.arange(pad, dtype=jnp.int32) % jnp.int32(N)
    pad_dst = jnp.full((pad,), N, dtype=jnp.int32)
    srcs = jnp.concatenate([src, pad_src]).reshape(C_TOT, K)
    dsts = jnp.concatenate([dst, pad_dst]).reshape(C_TOT, K)

    degp = _deg_kernel(dsts)
    h, dinv, hw = _pre_call(x[:, 0:1], x[:, 1:2], degp, classes_emb,
                            neigh_emb, Ws[0])

    for i in range(LAYERS - 1):
        part = _agg_kernel(hw, srcs, dsts)
        h, hw = _layer_call(h, hw, part, dinv,
                            bs[i].reshape(1, H), gammas[i].reshape(1, H),
                            betas[i].reshape(1, H), Ws[i + 1])

    part = _agg_kernel(hw, srcs, dsts)
    (h,) = _last_call(h, hw, part, dinv,
                      bs[3].reshape(1, H), gammas[3].reshape(1, H),
                      betas[3].reshape(1, H))

    roots = h.reshape(NB, N // NB, H)[:, 0, :]
    bz = (batch[-1] + 1 - NB).astype(jnp.float32).reshape(1, 1)
    return _logits_call(roots, Wout, bz)


# speedup vs baseline: 20.7111x; 1.7364x over previous
import functools

import jax
import jax.numpy as jnp
from jax import lax
from jax.experimental import pallas as pl
from jax.experimental.pallas import tpu as pltpu
from jax.experimental.pallas import tpu_sc as plsc

N = 10000
E = 320000
H = 128
IN_DIM = 32
NB = 100
LAYERS = 4

NC = 2
NS = 16
NW = NC * NS

K = 128
CW = 80
C_TOT = CW * NW
E_PAD = C_TOT * K
N_ACC = 10240
RPT = N_ACC // NS

_mesh = plsc.VectorSubcoreMesh(
    core_axis_name="c", subcore_axis_name="s", num_cores=NC, num_subcores=NS
)


def _zero_buf(buf, rows, width):
    def zrow(i, carry):
        for j in range(width // 16):
            buf[i, pl.ds(j * 16, 16)] = jnp.zeros((16,), jnp.float32)
        return carry
    lax.fori_loop(0, rows, zrow, 0)


def _fill_ones(buf, rows, width):
    def frow(i, carry):
        for j in range(width // 16):
            buf[i, pl.ds(j * 16, 16)] = jnp.ones((16,), jnp.float32)
        return carry
    lax.fori_loop(0, rows, frow, 0)


@functools.partial(
    pl.kernel,
    out_type=jax.ShapeDtypeStruct((NC, N_ACC, 16), jnp.float32),
    mesh=_mesh,
    scratch_types=[
        pltpu.VMEM((CW, K), jnp.int32),
        pltpu.VMEM((K, 16), jnp.float32),
        pltpu.VMEM_SHARED((N_ACC, 16), jnp.float32),
    ],
)
def _deg_kernel(dsts_hbm, out_hbm, dst_v, ones_v, acc):
    c = lax.axis_index("c")
    s = lax.axis_index("s")
    base = (c * NS + s) * CW
    pltpu.sync_copy(dsts_hbm.at[pl.ds(base, CW)], dst_v)
    _zero_buf(ones_v, K, 16)
    r0 = s * RPT
    for w in range(RPT // K):
        pltpu.sync_copy(ones_v, acc.at[pl.ds(r0 + w * K, K)])
    _fill_ones(ones_v, K, 16)
    plsc.subcore_barrier()

    def body(t, carry):
        pltpu.sync_copy(ones_v, acc.at[dst_v.at[t]], add=True)
        return carry

    lax.fori_loop(0, CW, body, 0)
    plsc.subcore_barrier()
    for w in range(RPT // K):
        sl = pl.ds(r0 + w * K, K)
        pltpu.sync_copy(acc.at[sl], out_hbm.at[c].at[sl])


@functools.partial(
    pl.kernel,
    out_type=jax.ShapeDtypeStruct((NC, N_ACC, H), jnp.float32),
    mesh=_mesh,
    scratch_types=[
        pltpu.VMEM((CW // 2, K), jnp.int32),
        pltpu.VMEM((CW // 2, K), jnp.int32),
        pltpu.VMEM((K, H), jnp.float32),
        pltpu.VMEM((K, H), jnp.float32),
        pltpu.VMEM_SHARED((N_ACC, H), jnp.float32),
        pltpu.SemaphoreType.DMA,
        pltpu.SemaphoreType.DMA,
    ],
)
def _agg_kernel(hw_hbm, srcs_hbm, dsts_hbm, out_hbm, src_v, dst_v,
                rows0, rows1, acc, sem0, sem1):
    c = lax.axis_index("c")
    s = lax.axis_index("s")
    base = (c * NS + s) * CW
    _zero_buf(rows0, K, H)
    r0 = s * RPT
    for w in range(RPT // K):
        pltpu.sync_copy(rows0, acc.at[pl.ds(r0 + w * K, K)])
    plsc.subcore_barrier()

    rows = (rows0, rows1)
    sems = (sem0, sem1)
    IW = CW // 2
    for phase in range(2):
        pltpu.sync_copy(srcs_hbm.at[pl.ds(base + phase * IW, IW)], src_v)
        pltpu.sync_copy(dsts_hbm.at[pl.ds(base + phase * IW, IW)], dst_v)
        pltpu.async_copy(hw_hbm.at[src_v.at[0]], rows0, sem0)

        def body(u, carry):
            for par in range(2):
                t = 2 * u + par
                pltpu.make_async_copy(
                    hw_hbm.at[src_v.at[t]], rows[par], sems[par]).wait()

                @pl.when(t + 1 < IW)
                def _():
                    pltpu.async_copy(
                        hw_hbm.at[src_v.at[t + 1]], rows[1 - par], sems[1 - par])

                pltpu.sync_copy(rows[par], acc.at[dst_v.at[t]], add=True)
            return carry

        lax.fori_loop(0, IW // 2, body, 0)
    plsc.subcore_barrier()
    for w in range(RPT // K):
        sl = pl.ds(r0 + w * K, K)
        pltpu.sync_copy(acc.at[sl], out_hbm.at[c].at[sl])


R = 1000
GRID = N // R


def _pre_body(x0_ref, x1_ref, degp_ref, ce_ref, ne_ref, w0_ref,
              h_ref, dinv_ref, hw_ref):
    x0 = x0_ref[...]
    x1 = x1_ref[...]
    oh0 = (x0 == lax.broadcasted_iota(jnp.int32, (R, IN_DIM), 1)).astype(jnp.float32)
    oh1 = (x1 == lax.broadcasted_iota(jnp.int32, (R, IN_DIM + 1), 1)).astype(jnp.float32)
    h = (jnp.dot(oh0, ce_ref[...], preferred_element_type=jnp.float32)
         + jnp.dot(oh1, ne_ref[...], preferred_element_type=jnp.float32))
    deg = degp_ref[0, :, 0:1] + degp_ref[1, :, 0:1] + 1.0
    dinv = lax.rsqrt(deg)
    h_ref[...] = h
    dinv_ref[...] = dinv
    hw_ref[...] = dinv * jnp.dot(h, w0_ref[...], preferred_element_type=jnp.float32)


_pre_call = pl.pallas_call(
    _pre_body,
    grid=(GRID,),
    in_specs=[
        pl.BlockSpec((R, 1), lambda i: (i, 0)),
        pl.BlockSpec((R, 1), lambda i: (i, 0)),
        pl.BlockSpec((NC, R, 16), lambda i: (0, i, 0)),
        pl.BlockSpec((IN_DIM, H), lambda i: (0, 0)),
        pl.BlockSpec((IN_DIM + 1, H), lambda i: (0, 0)),
        pl.BlockSpec((H, H), lambda i: (0, 0)),
    ],
    out_specs=[
        pl.BlockSpec((R, H), lambda i: (i, 0)),
        pl.BlockSpec((R, 1), lambda i: (i, 0)),
        pl.BlockSpec((R, H), lambda i: (i, 0)),
    ],
    out_shape=[
        jax.ShapeDtypeStruct((N, H), jnp.float32),
        jax.ShapeDtypeStruct((N, 1), jnp.float32),
        jax.ShapeDtypeStruct((N, H), jnp.float32),
    ],
)


def _layer_body(h_ref, hw_ref, p_ref, dinv_ref, b_ref, g_ref, be_ref, wn_ref,
                ho_ref, hwo_ref):
    dinv = dinv_ref[...]
    agg = p_ref[0] + p_ref[1] + hw_ref[...]
    conv = dinv * agg + b_ref[...]
    hn = jnp.maximum(conv, 0.0)
    sres = h_ref[...] + hn
    m = jnp.mean(sres, axis=-1, keepdims=True)
    d = sres - m
    v = jnp.mean(d * d, axis=-1, keepdims=True)
    hnew = d * lax.rsqrt(v + 1e-5) * g_ref[...] + be_ref[...]
    ho_ref[...] = hnew
    hwo_ref[...] = dinv * jnp.dot(hnew, wn_ref[...], preferred_element_type=jnp.float32)


_layer_call = pl.pallas_call(
    _layer_body,
    grid=(GRID,),
    in_specs=[
        pl.BlockSpec((R, H), lambda i: (i, 0)),
        pl.BlockSpec((R, H), lambda i: (i, 0)),
        pl.BlockSpec((NC, R, H), lambda i: (0, i, 0)),
        pl.BlockSpec((R, 1), lambda i: (i, 0)),
        pl.BlockSpec((1, H), lambda i: (0, 0)),
        pl.BlockSpec((1, H), lambda i: (0, 0)),
        pl.BlockSpec((1, H), lambda i: (0, 0)),
        pl.BlockSpec((H, H), lambda i: (0, 0)),
    ],
    out_specs=[
        pl.BlockSpec((R, H), lambda i: (i, 0)),
        pl.BlockSpec((R, H), lambda i: (i, 0)),
    ],
    out_shape=[
        jax.ShapeDtypeStruct((N, H), jnp.float32),
        jax.ShapeDtypeStruct((N, H), jnp.float32),
    ],
)


def _last_body(h_ref, hw_ref, p_ref, dinv_ref, b_ref, g_ref, be_ref, ho_ref):
    dinv = dinv_ref[...]
    agg = p_ref[0] + p_ref[1] + hw_ref[...]
    conv = dinv * agg + b_ref[...]
    hn = jnp.maximum(conv, 0.0)
    sres = h_ref[...] + hn
    m = jnp.mean(sres, axis=-1, keepdims=True)
    d = sres - m
    v = jnp.mean(d * d, axis=-1, keepdims=True)
    ho_ref[...] = d * lax.rsqrt(v + 1e-5) * g_ref[...] + be_ref[...]


_last_call = pl.pallas_call(
    _last_body,
    grid=(GRID,),
    in_specs=[
        pl.BlockSpec((R, H), lambda i: (i, 0)),
        pl.BlockSpec((R, H), lambda i: (i, 0)),
        pl.BlockSpec((NC, R, H), lambda i: (0, i, 0)),
        pl.BlockSpec((R, 1), lambda i: (i, 0)),
        pl.BlockSpec((1, H), lambda i: (0, 0)),
        pl.BlockSpec((1, H), lambda i: (0, 0)),
        pl.BlockSpec((1, H), lambda i: (0, 0)),
    ],
    out_specs=[pl.BlockSpec((R, H), lambda i: (i, 0))],
    out_shape=[jax.ShapeDtypeStruct((N, H), jnp.float32)],
)


def _logits_body(roots_ref, wout_ref, bz_ref, out_ref):
    out_ref[...] = (
        jnp.dot(roots_ref[...], wout_ref[...], preferred_element_type=jnp.float32)
        + bz_ref[...]
    )


_logits_call = pl.pallas_call(
    _logits_body,
    out_shape=jax.ShapeDtypeStruct((NB, IN_DIM), jnp.float32),
)


def kernel(x, edge_index, batch, classes_emb, neigh_emb, Ws, bs, gammas, betas, Wout):
    src = edge_index[0]
    dst = edge_index[1]
    pad = E_PAD - E
    pad_src = jnp.arange(pad, dtype=jnp.int32) % jnp.int32(N)
    pad_dst = jnp.full((pad,), N, dtype=jnp.int32)
    srcs = jnp.concatenate([src, pad_src]).reshape(C_TOT, K)
    dsts = jnp.concatenate([dst, pad_dst]).reshape(C_TOT, K)

    degp = _deg_kernel(dsts)
    h, dinv, hw = _pre_call(x[:, 0:1], x[:, 1:2], degp, classes_emb,
                            neigh_emb, Ws[0])

    for i in range(LAYERS - 1):
        part = _agg_kernel(hw, srcs, dsts)
        h, hw = _layer_call(h, hw, part, dinv,
                            bs[i].reshape(1, H), gammas[i].reshape(1, H),
                            betas[i].reshape(1, H), Ws[i + 1])

    part = _agg_kernel(hw, srcs, dsts)
    (h,) = _last_call(h, hw, part, dinv,
                      bs[3].reshape(1, H), gammas[3].reshape(1, H),
                      betas[3].reshape(1, H))

    roots = h.reshape(NB, N // NB, H)[:, 0, :]
    bz = (batch[-1] + 1 - NB).astype(jnp.float32).reshape(1, 1)
    return _logits_call(roots, Wout, bz)


# speedup vs baseline: 20.7283x; 1.0008x over previous
import functools

import jax
import jax.numpy as jnp
from jax import lax
from jax.experimental import pallas as pl
from jax.experimental.pallas import tpu as pltpu
from jax.experimental.pallas import tpu_sc as plsc

N = 10000
E = 320000
H = 128
IN_DIM = 32
NB = 100
LAYERS = 4

NC = 2
NS = 16
NW = NC * NS

K = 128
CW = 80
C_TOT = CW * NW
E_PAD = C_TOT * K
N_ACC = 10240
RPT = N_ACC // NS

_mesh = plsc.VectorSubcoreMesh(
    core_axis_name="c", subcore_axis_name="s", num_cores=NC, num_subcores=NS
)


def _zero_buf(buf, rows, width):
    def zrow(i, carry):
        for j in range(width // 16):
            buf[i, pl.ds(j * 16, 16)] = jnp.zeros((16,), jnp.float32)
        return carry
    lax.fori_loop(0, rows, zrow, 0)


def _fill_ones(buf, rows, width):
    def frow(i, carry):
        for j in range(width // 16):
            buf[i, pl.ds(j * 16, 16)] = jnp.ones((16,), jnp.float32)
        return carry
    lax.fori_loop(0, rows, frow, 0)


@functools.partial(
    pl.kernel,
    out_type=jax.ShapeDtypeStruct((NC, N_ACC, 16), jnp.float32),
    mesh=_mesh,
    scratch_types=[
        pltpu.VMEM((CW, K), jnp.int32),
        pltpu.VMEM((K, 16), jnp.float32),
        pltpu.VMEM_SHARED((N_ACC, 16), jnp.float32),
    ],
)
def _deg_kernel(dsts_hbm, out_hbm, dst_v, ones_v, acc):
    c = lax.axis_index("c")
    s = lax.axis_index("s")
    base = (c * NS + s) * CW
    pltpu.sync_copy(dsts_hbm.at[pl.ds(base, CW)], dst_v)
    _zero_buf(ones_v, K, 16)
    r0 = s * RPT
    for w in range(RPT // K):
        pltpu.sync_copy(ones_v, acc.at[pl.ds(r0 + w * K, K)])
    _fill_ones(ones_v, K, 16)
    plsc.subcore_barrier()

    def body(t, carry):
        pltpu.sync_copy(ones_v, acc.at[dst_v.at[t]], add=True)
        return carry

    lax.fori_loop(0, CW, body, 0)
    plsc.subcore_barrier()
    for w in range(RPT // K):
        sl = pl.ds(r0 + w * K, K)
        pltpu.sync_copy(acc.at[sl], out_hbm.at[c].at[sl])


IW = CW // 2


@functools.partial(
    pl.kernel,
    out_type=jax.ShapeDtypeStruct((NC, N_ACC, H), jnp.float32),
    mesh=_mesh,
    scratch_types=[
        pltpu.VMEM((IW, K), jnp.int32),
        pltpu.VMEM((IW, K), jnp.int32),
        pltpu.VMEM((K, H), jnp.float32),
        pltpu.VMEM((K, H), jnp.float32),
        pltpu.VMEM_SHARED((N_ACC, H), jnp.float32),
        pltpu.SemaphoreType.DMA,
        pltpu.SemaphoreType.DMA,
        pltpu.SemaphoreType.DMA,
        pltpu.SemaphoreType.DMA,
    ],
)
def _agg_kernel(hw_hbm, srcs_hbm, dsts_hbm, out_hbm, src_v, dst_v,
                rows0, rows1, acc, gsem0, gsem1, ssem0, ssem1):
    c = lax.axis_index("c")
    s = lax.axis_index("s")
    base = (c * NS + s) * CW
    rows = (rows0, rows1)
    gsems = (gsem0, gsem1)
    ssems = (ssem0, ssem1)

    _zero_buf(rows0, K, H)
    r0 = s * RPT
    for w in range(RPT // K):
        pltpu.sync_copy(rows0, acc.at[pl.ds(r0 + w * K, K)])
    plsc.subcore_barrier()

    for phase in range(2):
        if phase == 1:
            pltpu.make_async_copy(rows1, acc.at[dst_v.at[IW - 1]], ssem1).wait()
        pltpu.sync_copy(srcs_hbm.at[pl.ds(base + phase * IW, IW)], src_v)
        pltpu.sync_copy(dsts_hbm.at[pl.ds(base + phase * IW, IW)], dst_v)
        pltpu.async_copy(hw_hbm.at[src_v.at[0]], rows0, gsem0)

        pltpu.make_async_copy(hw_hbm.at[src_v.at[0]], rows0, gsem0).wait()
        pltpu.async_copy(rows0, acc.at[dst_v.at[0]], ssem0, add=True)
        pltpu.async_copy(hw_hbm.at[src_v.at[1]], rows1, gsem1)

        def body(u, carry):
            for off in (1, 2):
                t = 2 * u + off
                par = off % 2
                pltpu.make_async_copy(
                    hw_hbm.at[src_v.at[t]], rows[par], gsems[par]).wait()
                pltpu.make_async_copy(
                    rows[1 - par], acc.at[dst_v.at[t - 1]], ssems[1 - par]).wait()
                pltpu.async_copy(rows[par], acc.at[dst_v.at[t]], ssems[par],
                                 add=True)
                pltpu.async_copy(
                    hw_hbm.at[src_v.at[t + 1]], rows[1 - par], gsems[1 - par])
            return carry

        lax.fori_loop(0, (IW - 2) // 2, body, 0)

        pltpu.make_async_copy(hw_hbm.at[src_v.at[IW - 1]], rows1, gsem1).wait()
        pltpu.make_async_copy(rows0, acc.at[dst_v.at[IW - 2]], ssem0).wait()
        pltpu.async_copy(rows1, acc.at[dst_v.at[IW - 1]], ssem1, add=True)

    pltpu.make_async_copy(rows1, acc.at[dst_v.at[IW - 1]], ssem1).wait()
    plsc.subcore_barrier()
    for w in range(RPT // K):
        sl = pl.ds(r0 + w * K, K)
        pltpu.sync_copy(acc.at[sl], out_hbm.at[c].at[sl])


R = 1000
GRID = N // R


def _pre_body(x0_ref, x1_ref, degp_ref, ce_ref, ne_ref, w0_ref,
              h_ref, dinv_ref, hw_ref):
    x0 = x0_ref[...]
    x1 = x1_ref[...]
    oh0 = (x0 == lax.broadcasted_iota(jnp.int32, (R, IN_DIM), 1)).astype(jnp.float32)
    oh1 = (x1 == lax.broadcasted_iota(jnp.int32, (R, IN_DIM + 1), 1)).astype(jnp.float32)
    h = (jnp.dot(oh0, ce_ref[...], preferred_element_type=jnp.float32)
         + jnp.dot(oh1, ne_ref[...], preferred_element_type=jnp.float32))
    deg = degp_ref[0, :, 0:1] + degp_ref[1, :, 0:1] + 1.0
    dinv = lax.rsqrt(deg)
    h_ref[...] = h
    dinv_ref[...] = dinv
    hw_ref[...] = dinv * jnp.dot(h, w0_ref[...], preferred_element_type=jnp.float32)


_pre_call = pl.pallas_call(
    _pre_body,
    grid=(GRID,),
    in_specs=[
        pl.BlockSpec((R, 1), lambda i: (i, 0)),
        pl.BlockSpec((R, 1), lambda i: (i, 0)),
        pl.BlockSpec((NC, R, 16), lambda i: (0, i, 0)),
        pl.BlockSpec((IN_DIM, H), lambda i: (0, 0)),
        pl.BlockSpec((IN_DIM + 1, H), lambda i: (0, 0)),
        pl.BlockSpec((H, H), lambda i: (0, 0)),
    ],
    out_specs=[
        pl.BlockSpec((R, H), lambda i: (i, 0)),
        pl.BlockSpec((R, 1), lambda i: (i, 0)),
        pl.BlockSpec((R, H), lambda i: (i, 0)),
    ],
    out_shape=[
        jax.ShapeDtypeStruct((N, H), jnp.float32),
        jax.ShapeDtypeStruct((N, 1), jnp.float32),
        jax.ShapeDtypeStruct((N, H), jnp.float32),
    ],
)


def _ln_block(h_ref, hw_ref, p_ref, dinv_ref, b_ref, g_ref, be_ref):
    dinv = dinv_ref[...]
    agg = p_ref[0] + p_ref[1] + hw_ref[...]
    conv = dinv * agg + b_ref[...]
    hn = jnp.maximum(conv, 0.0)
    sres = h_ref[...] + hn
    m = jnp.mean(sres, axis=-1, keepdims=True)
    d = sres - m
    v = jnp.mean(d * d, axis=-1, keepdims=True)
    return d * lax.rsqrt(v + 1e-5) * g_ref[...] + be_ref[...]


def _layer_body(h_ref, hw_ref, p_ref, dinv_ref, b_ref, g_ref, be_ref, wn_ref,
                ho_ref, hwo_ref):
    hnew = _ln_block(h_ref, hw_ref, p_ref, dinv_ref, b_ref, g_ref, be_ref)
    ho_ref[...] = hnew
    hwo_ref[...] = dinv_ref[...] * jnp.dot(hnew, wn_ref[...],
                                           preferred_element_type=jnp.float32)


_layer_call = pl.pallas_call(
    _layer_body,
    grid=(GRID,),
    in_specs=[
        pl.BlockSpec((R, H), lambda i: (i, 0)),
        pl.BlockSpec((R, H), lambda i: (i, 0)),
        pl.BlockSpec((NC, R, H), lambda i: (0, i, 0)),
        pl.BlockSpec((R, 1), lambda i: (i, 0)),
        pl.BlockSpec((1, H), lambda i: (0, 0)),
        pl.BlockSpec((1, H), lambda i: (0, 0)),
        pl.BlockSpec((1, H), lambda i: (0, 0)),
        pl.BlockSpec((H, H), lambda i: (0, 0)),
    ],
    out_specs=[
        pl.BlockSpec((R, H), lambda i: (i, 0)),
        pl.BlockSpec((R, H), lambda i: (i, 0)),
    ],
    out_shape=[
        jax.ShapeDtypeStruct((N, H), jnp.float32),
        jax.ShapeDtypeStruct((N, H), jnp.float32),
    ],
)


def _last_body(h_ref, hw_ref, p_ref, dinv_ref, b_ref, g_ref, be_ref, ho_ref):
    ho_ref[...] = _ln_block(h_ref, hw_ref, p_ref, dinv_ref, b_ref, g_ref, be_ref)


_last_call = pl.pallas_call(
    _last_body,
    grid=(GRID,),
    in_specs=[
        pl.BlockSpec((R, H), lambda i: (i, 0)),
        pl.BlockSpec((R, H), lambda i: (i, 0)),
        pl.BlockSpec((NC, R, H), lambda i: (0, i, 0)),
        pl.BlockSpec((R, 1), lambda i: (i, 0)),
        pl.BlockSpec((1, H), lambda i: (0, 0)),
        pl.BlockSpec((1, H), lambda i: (0, 0)),
        pl.BlockSpec((1, H), lambda i: (0, 0)),
    ],
    out_specs=[pl.BlockSpec((R, H), lambda i: (i, 0))],
    out_shape=[jax.ShapeDtypeStruct((N, H), jnp.float32)],
)


def _logits_body(roots_ref, wout_ref, bz_ref, out_ref):
    out_ref[...] = (
        jnp.dot(roots_ref[...], wout_ref[...], preferred_element_type=jnp.float32)
        + bz_ref[...]
    )


_logits_call = pl.pallas_call(
    _logits_body,
    out_shape=jax.ShapeDtypeStruct((NB, IN_DIM), jnp.float32),
)


def kernel(x, edge_index, batch, classes_emb, neigh_emb, Ws, bs, gammas, betas, Wout):
    src = edge_index[0]
    dst = edge_index[1]
    pad = E_PAD - E
    pad_src = jnp.arange(pad, dtype=jnp.int32) % jnp.int32(N)
    pad_dst = jnp.full((pad,), N, dtype=jnp.int32)
    srcs = jnp.concatenate([src, pad_src]).reshape(C_TOT, K)
    dsts = jnp.concatenate([dst, pad_dst]).reshape(C_TOT, K)

    degp = _deg_kernel(dsts)
    h, dinv, hw = _pre_call(x[:, 0:1], x[:, 1:2], degp, classes_emb,
                            neigh_emb, Ws[0])

    for i in range(LAYERS - 1):
        part = _agg_kernel(hw, srcs, dsts)
        h, hw = _layer_call(h, hw, part, dinv,
                            bs[i].reshape(1, H), gammas[i].reshape(1, H),
                            betas[i].reshape(1, H), Ws[i + 1])

    part = _agg_kernel(hw, srcs, dsts)
    (h,) = _last_call(h, hw, part, dinv,
                      bs[3].reshape(1, H), gammas[3].reshape(1, H),
                      betas[3].reshape(1, H))

    roots = h.reshape(NB, N // NB, H)[:, 0, :]
    bz = (batch[-1] + 1 - NB).astype(jnp.float32).reshape(1, 1)
    return _logits_call(roots, Wout, bz)


# speedup vs baseline: 20.9005x; 1.0083x over previous
import functools

import jax
import jax.numpy as jnp
from jax import lax
from jax.experimental import pallas as pl
from jax.experimental.pallas import tpu as pltpu
from jax.experimental.pallas import tpu_sc as plsc

N = 10000
E = 320000
H = 128
IN_DIM = 32
NB = 100
LAYERS = 4

NC = 2
NS = 16
NW = NC * NS

K = 128
CW = 80
C_TOT = CW * NW
E_PAD = C_TOT * K
N_ACC = 10240
RPT = N_ACC // NS

_mesh = plsc.VectorSubcoreMesh(
    core_axis_name="c", subcore_axis_name="s", num_cores=NC, num_subcores=NS
)


def _zero_buf(buf, rows, width):
    def zrow(i, carry):
        for j in range(width // 16):
            buf[i, pl.ds(j * 16, 16)] = jnp.zeros((16,), jnp.float32)
        return carry
    lax.fori_loop(0, rows, zrow, 0)


def _fill_ones(buf, rows, width):
    def frow(i, carry):
        for j in range(width // 16):
            buf[i, pl.ds(j * 16, 16)] = jnp.ones((16,), jnp.float32)
        return carry
    lax.fori_loop(0, rows, frow, 0)


@functools.partial(
    pl.kernel,
    out_type=jax.ShapeDtypeStruct((NC, N_ACC, 16), jnp.float32),
    mesh=_mesh,
    scratch_types=[
        pltpu.VMEM((CW, K), jnp.int32),
        pltpu.VMEM((K, 16), jnp.float32),
        pltpu.VMEM_SHARED((N_ACC, 16), jnp.float32),
    ],
)
def _deg_kernel(dsts_hbm, out_hbm, dst_v, ones_v, acc):
    c = lax.axis_index("c")
    s = lax.axis_index("s")
    base = (c * NS + s) * CW
    pltpu.sync_copy(dsts_hbm.at[pl.ds(base, CW)], dst_v)
    _zero_buf(ones_v, K, 16)
    r0 = s * RPT
    for w in range(RPT // K):
        pltpu.sync_copy(ones_v, acc.at[pl.ds(r0 + w * K, K)])
    _fill_ones(ones_v, K, 16)
    plsc.subcore_barrier()

    def body(t, carry):
        pltpu.sync_copy(ones_v, acc.at[dst_v.at[t]], add=True)
        return carry

    lax.fori_loop(0, CW, body, 0)
    plsc.subcore_barrier()
    for w in range(RPT // K):
        sl = pl.ds(r0 + w * K, K)
        pltpu.sync_copy(acc.at[sl], out_hbm.at[c].at[sl])


IW = CW // 2


@functools.partial(
    pl.kernel,
    out_type=jax.ShapeDtypeStruct((NC, N_ACC, H), jnp.float32),
    mesh=_mesh,
    scratch_types=[
        pltpu.VMEM((IW, K), jnp.int32),
        pltpu.VMEM((IW, K), jnp.int32),
        pltpu.VMEM((K, H), jnp.float32),
        pltpu.VMEM((K, H), jnp.float32),
        pltpu.VMEM_SHARED((N_ACC, H), jnp.float32),
        pltpu.SemaphoreType.DMA,
        pltpu.SemaphoreType.DMA,
        pltpu.SemaphoreType.DMA,
        pltpu.SemaphoreType.DMA,
    ],
)
def _agg_kernel(hw_hbm, srcs_hbm, dsts_hbm, out_hbm, src_v, dst_v,
                rows0, rows1, acc, gsem0, gsem1, ssem0, ssem1):
    c = lax.axis_index("c")
    s = lax.axis_index("s")
    base = (c * NS + s) * CW
    rows = (rows0, rows1)
    gsems = (gsem0, gsem1)
    ssems = (ssem0, ssem1)

    _zero_buf(rows0, K, H)
    r0 = s * RPT
    for w in range(RPT // K):
        pltpu.sync_copy(rows0, acc.at[pl.ds(r0 + w * K, K)])
    plsc.subcore_barrier()

    for phase in range(2):
        if phase == 1:
            pltpu.make_async_copy(rows1, acc.at[dst_v.at[IW - 1]], ssem1).wait()
        pltpu.sync_copy(srcs_hbm.at[pl.ds(base + phase * IW, IW)], src_v)
        pltpu.sync_copy(dsts_hbm.at[pl.ds(base + phase * IW, IW)], dst_v)
        pltpu.async_copy(hw_hbm.at[src_v.at[0]], rows0, gsem0)

        pltpu.make_async_copy(hw_hbm.at[src_v.at[0]], rows0, gsem0).wait()
        pltpu.async_copy(rows0, acc.at[dst_v.at[0]], ssem0, add=True)
        pltpu.async_copy(hw_hbm.at[src_v.at[1]], rows1, gsem1)

        def body(u, carry):
            for off in (1, 2):
                t = 2 * u + off
                par = off % 2
                pltpu.make_async_copy(
                    hw_hbm.at[src_v.at[t]], rows[par], gsems[par]).wait()
                pltpu.make_async_copy(
                    rows[1 - par], acc.at[dst_v.at[t - 1]], ssems[1 - par]).wait()
                pltpu.async_copy(rows[par], acc.at[dst_v.at[t]], ssems[par],
                                 add=True)
                pltpu.async_copy(
                    hw_hbm.at[src_v.at[t + 1]], rows[1 - par], gsems[1 - par])
            return carry

        lax.fori_loop(0, (IW - 2) // 2, body, 0)

        pltpu.make_async_copy(hw_hbm.at[src_v.at[IW - 1]], rows1, gsem1).wait()
        pltpu.make_async_copy(rows0, acc.at[dst_v.at[IW - 2]], ssem0).wait()
        pltpu.async_copy(rows1, acc.at[dst_v.at[IW - 1]], ssem1, add=True)

    pltpu.make_async_copy(rows1, acc.at[dst_v.at[IW - 1]], ssem1).wait()
    plsc.subcore_barrier()
    for w in range(RPT // K):
        sl = pl.ds(r0 + w * K, K)
        pltpu.sync_copy(acc.at[sl], out_hbm.at[c].at[sl])


R = 1000
GRID = N // R


def _embed_body(x0_ref, x1_ref, ce_ref, ne_ref, h_ref):
    x0 = x0_ref[...]
    x1 = x1_ref[...]
    oh0 = (x0 == lax.broadcasted_iota(jnp.int32, (R, IN_DIM), 1)).astype(jnp.float32)
    oh1 = (x1 == lax.broadcasted_iota(jnp.int32, (R, IN_DIM + 1), 1)).astype(jnp.float32)
    h_ref[...] = (jnp.dot(oh0, ce_ref[...], preferred_element_type=jnp.float32)
                  + jnp.dot(oh1, ne_ref[...], preferred_element_type=jnp.float32))


_embed_call = pl.pallas_call(
    _embed_body,
    grid=(GRID,),
    in_specs=[
        pl.BlockSpec((R, 1), lambda i: (i, 0)),
        pl.BlockSpec((R, 1), lambda i: (i, 0)),
        pl.BlockSpec((IN_DIM, H), lambda i: (0, 0)),
        pl.BlockSpec((IN_DIM + 1, H), lambda i: (0, 0)),
    ],
    out_specs=pl.BlockSpec((R, H), lambda i: (i, 0)),
    out_shape=jax.ShapeDtypeStruct((N, H), jnp.float32),
)


def _pre_body(h_ref, degp_ref, w0_ref, dinv_ref, hw_ref):
    deg = degp_ref[0, :, 0:1] + degp_ref[1, :, 0:1] + 1.0
    dinv = lax.rsqrt(deg)
    dinv_ref[...] = dinv
    hw_ref[...] = dinv * jnp.dot(h_ref[...], w0_ref[...],
                                 preferred_element_type=jnp.float32)


_pre_call = pl.pallas_call(
    _pre_body,
    grid=(GRID,),
    in_specs=[
        pl.BlockSpec((R, H), lambda i: (i, 0)),
        pl.BlockSpec((NC, R, 16), lambda i: (0, i, 0)),
        pl.BlockSpec((H, H), lambda i: (0, 0)),
    ],
    out_specs=[
        pl.BlockSpec((R, 1), lambda i: (i, 0)),
        pl.BlockSpec((R, H), lambda i: (i, 0)),
    ],
    out_shape=[
        jax.ShapeDtypeStruct((N, 1), jnp.float32),
        jax.ShapeDtypeStruct((N, H), jnp.float32),
    ],
)


def _ln_block(h_ref, hw_ref, p_ref, dinv_ref, b_ref, g_ref, be_ref):
    dinv = dinv_ref[...]
    agg = p_ref[0] + p_ref[1] + hw_ref[...]
    conv = dinv * agg + b_ref[...]
    hn = jnp.maximum(conv, 0.0)
    sres = h_ref[...] + hn
    m = jnp.mean(sres, axis=-1, keepdims=True)
    d = sres - m
    v = jnp.mean(d * d, axis=-1, keepdims=True)
    return d * lax.rsqrt(v + 1e-5) * g_ref[...] + be_ref[...]


def _layer_body(h_ref, hw_ref, p_ref, dinv_ref, b_ref, g_ref, be_ref, wn_ref,
                ho_ref, hwo_ref):
    hnew = _ln_block(h_ref, hw_ref, p_ref, dinv_ref, b_ref, g_ref, be_ref)
    ho_ref[...] = hnew
    hwo_ref[...] = dinv_ref[...] * jnp.dot(hnew, wn_ref[...],
                                           preferred_element_type=jnp.float32)


_layer_call = pl.pallas_call(
    _layer_body,
    grid=(GRID,),
    in_specs=[
        pl.BlockSpec((R, H), lambda i: (i, 0)),
        pl.BlockSpec((R, H), lambda i: (i, 0)),
        pl.BlockSpec((NC, R, H), lambda i: (0, i, 0)),
        pl.BlockSpec((R, 1), lambda i: (i, 0)),
        pl.BlockSpec((1, H), lambda i: (0, 0)),
        pl.BlockSpec((1, H), lambda i: (0, 0)),
        pl.BlockSpec((1, H), lambda i: (0, 0)),
        pl.BlockSpec((H, H), lambda i: (0, 0)),
    ],
    out_specs=[
        pl.BlockSpec((R, H), lambda i: (i, 0)),
        pl.BlockSpec((R, H), lambda i: (i, 0)),
    ],
    out_shape=[
        jax.ShapeDtypeStruct((N, H), jnp.float32),
        jax.ShapeDtypeStruct((N, H), jnp.float32),
    ],
)


def _last_body(h_ref, hw_ref, p_ref, dinv_ref, b_ref, g_ref, be_ref, ho_ref):
    ho_ref[...] = _ln_block(h_ref, hw_ref, p_ref, dinv_ref, b_ref, g_ref, be_ref)


_last_call = pl.pallas_call(
    _last_body,
    grid=(GRID,),
    in_specs=[
        pl.BlockSpec((R, H), lambda i: (i, 0)),
        pl.BlockSpec((R, H), lambda i: (i, 0)),
        pl.BlockSpec((NC, R, H), lambda i: (0, i, 0)),
        pl.BlockSpec((R, 1), lambda i: (i, 0)),
        pl.BlockSpec((1, H), lambda i: (0, 0)),
        pl.BlockSpec((1, H), lambda i: (0, 0)),
        pl.BlockSpec((1, H), lambda i: (0, 0)),
    ],
    out_specs=[pl.BlockSpec((R, H), lambda i: (i, 0))],
    out_shape=[jax.ShapeDtypeStruct((N, H), jnp.float32)],
)


def _logits_body(roots_ref, wout_ref, bz_ref, out_ref):
    out_ref[...] = (
        jnp.dot(roots_ref[...], wout_ref[...], preferred_element_type=jnp.float32)
        + bz_ref[...]
    )


_logits_call = pl.pallas_call(
    _logits_body,
    out_shape=jax.ShapeDtypeStruct((NB, IN_DIM), jnp.float32),
)


def kernel(x, edge_index, batch, classes_emb, neigh_emb, Ws, bs, gammas, betas, Wout):
    src = edge_index[0]
    dst = edge_index[1]
    pad = E_PAD - E
    pad_src = jnp.arange(pad, dtype=jnp.int32) % jnp.int32(N)
    pad_dst = N + jnp.arange(pad, dtype=jnp.int32) % jnp.int32(N_ACC - N)
    srcs = jnp.concatenate([src, pad_src]).reshape(C_TOT, K)
    dsts = jnp.concatenate([dst, pad_dst]).reshape(C_TOT, K)

    degp = _deg_kernel(dsts)
    h = _embed_call(x[:, 0:1], x[:, 1:2], classes_emb, neigh_emb)
    dinv, hw = _pre_call(h, degp, Ws[0])

    for i in range(LAYERS - 1):
        part = _agg_kernel(hw, srcs, dsts)
        h, hw = _layer_call(h, hw, part, dinv,
                            bs[i].reshape(1, H), gammas[i].reshape(1, H),
                            betas[i].reshape(1, H), Ws[i + 1])

    part = _agg_kernel(hw, srcs, dsts)
    (h,) = _last_call(h, hw, part, dinv,
                      bs[3].reshape(1, H), gammas[3].reshape(1, H),
                      betas[3].reshape(1, H))

    roots = h.reshape(NB, N // NB, H)[:, 0, :]
    bz = (batch[-1] + 1 - NB).astype(jnp.float32).reshape(1, 1)
    return _logits_call(roots, Wout, bz)
